# baseline (speedup 1.0000x reference)
import functools

import jax
import jax.numpy as jnp
from jax import lax
from jax.experimental import pallas as pl
from jax.experimental.pallas import tpu as pltpu
from jax.experimental.pallas import tpu_sc as plsc

N = 10000
E = 320000
D = 128

ROWS = 1000
NB = N // ROWS

NC = 2
NS = 16
NW = NC * NS
EW = E // NW
C = 50
NCHUNK = EW // C
SUB = 25
NBUF = 5
NSUP = NCHUNK // SUB
NPAD = 10240
SLAB = NPAD // NS

_MESH = plsc.VectorSubcoreMesh(core_axis_name="c", subcore_axis_name="s")


def _k1_body(f_ref, w1_ref, b1_ref, wga_ref, wgb_ref, nf_ref, xw_ref):
    f = f_ref[...]
    h = jnp.dot(f, w1_ref[...], preferred_element_type=jnp.float32) + b1_ref[...]
    nf = jnp.where(h > 0, h, jnp.exp(h) - 1.0)
    nf_ref[...] = nf
    xw_ref[...] = (jnp.dot(nf, wga_ref[...], preferred_element_type=jnp.float32)
                   + jnp.dot(f, wgb_ref[...], preferred_element_type=jnp.float32))


_k1 = pl.pallas_call(
    _k1_body,
    grid=(NB,),
    in_specs=[
        pl.BlockSpec((ROWS, D), lambda i: (i, 0)),
        pl.BlockSpec((D, D), lambda i: (0, 0)),
        pl.BlockSpec((1, D), lambda i: (0, 0)),
        pl.BlockSpec((D, D), lambda i: (0, 0)),
        pl.BlockSpec((D, D), lambda i: (0, 0)),
    ],
    out_specs=[
        pl.BlockSpec((ROWS, D), lambda i: (i, 0)),
        pl.BlockSpec((ROWS, D), lambda i: (i, 0)),
    ],
    out_shape=[
        jax.ShapeDtypeStruct((N, D), jnp.float32),
        jax.ShapeDtypeStruct((N, D), jnp.float32),
    ],
)


@functools.partial(
    pl.kernel,
    out_type=[jax.ShapeDtypeStruct((NPAD,), jnp.float32),
              jax.ShapeDtypeStruct((NPAD,), jnp.float32)],
    mesh=_MESH,
    scratch_types=[
        pltpu.VMEM((SUB, C), jnp.int32),
        pltpu.VMEM((64,), jnp.float32),
        pltpu.VMEM((SLAB,), jnp.float32),
        pltpu.VMEM_SHARED((NPAD,), jnp.float32),
    ],
)
def _sc_deg(dst_hbm, out0_hbm, out1_hbm, didx_v, ones_v, zbuf_v, deg_sh):
    c = lax.axis_index("c")
    s = lax.axis_index("s")
    wid = c * NS + s

    def fill_ones(i, _):
        ones_v[pl.ds(i * 16, 16)] = jnp.full((16,), 1.0, jnp.float32)
        return 0

    lax.fori_loop(0, 64 // 16, fill_ones, 0)

    def fill_zero(i, _):
        zbuf_v[pl.ds(i * 16, 16)] = jnp.zeros((16,), jnp.float32)
        return 0

    lax.fori_loop(0, SLAB // 16, fill_zero, 0)

    pltpu.sync_copy(zbuf_v, deg_sh.at[pl.ds(s * SLAB, SLAB)])
    plsc.subcore_barrier()

    def sup(k, _):
        pltpu.sync_copy(dst_hbm.at[wid, k], didx_v)

        def body(j, _):
            pltpu.sync_copy(ones_v.at[pl.ds(0, C)], deg_sh.at[didx_v.at[j]],
                            add=True)
            return 0

        lax.fori_loop(0, SUB, body, 0)
        return 0

    lax.fori_loop(0, NSUP, sup, 0)
    plsc.subcore_barrier()

    @pl.when(c == 0)
    def _():
        pltpu.sync_copy(deg_sh.at[pl.ds(s * SLAB, SLAB)],
                        out0_hbm.at[pl.ds(s * SLAB, SLAB)])

    @pl.when(c == 1)
    def _():
        pltpu.sync_copy(deg_sh.at[pl.ds(s * SLAB, SLAB)],
                        out1_hbm.at[pl.ds(s * SLAB, SLAB)])


def _k2_body(d0_ref, d1_ref, xw_ref, xws_ref, dinv_ref):
    deg = d0_ref[...] + d1_ref[...] + 1.0
    dinv = lax.rsqrt(deg)
    dinv_ref[...] = dinv
    xws_ref[...] = xw_ref[...] * dinv


_k2 = pl.pallas_call(
    _k2_body,
    grid=(NB,),
    in_specs=[
        pl.BlockSpec((ROWS, 1), lambda i: (i, 0)),
        pl.BlockSpec((ROWS, 1), lambda i: (i, 0)),
        pl.BlockSpec((ROWS, D), lambda i: (i, 0)),
    ],
    out_specs=[
        pl.BlockSpec((ROWS, D), lambda i: (i, 0)),
        pl.BlockSpec((ROWS, 1), lambda i: (i, 0)),
    ],
    out_shape=[
        jax.ShapeDtypeStruct((N, D), jnp.float32),
        jax.ShapeDtypeStruct((N, 1), jnp.float32),
    ],
)


@functools.partial(
    pl.kernel,
    out_type=[jax.ShapeDtypeStruct((NPAD, D), jnp.float32),
              jax.ShapeDtypeStruct((NPAD, D), jnp.float32)],
    mesh=_MESH,
    scratch_types=[
        pltpu.VMEM((SUB, C), jnp.int32),
        pltpu.VMEM((SUB, C), jnp.int32),
        pltpu.VMEM((NBUF, C, D), jnp.float32),
        pltpu.VMEM_SHARED((NPAD, D), jnp.float32),
    ] + [pltpu.SemaphoreType.DMA] * (2 * NBUF),
)
def _sc_agg(src_hbm, dst_hbm, xws_hbm, zeros_hbm, out0_hbm, out1_hbm,
            sidx_v, didx_v, rows_v, agg_sh, *sems):
    gsem = sems[:NBUF]
    ssem = sems[NBUF:]
    c = lax.axis_index("c")
    s = lax.axis_index("s")
    wid = c * NS + s

    pltpu.sync_copy(zeros_hbm.at[pl.ds(s * SLAB, SLAB)],
                    agg_sh.at[pl.ds(s * SLAB, SLAB)])
    plsc.subcore_barrier()

    def gath(j, b):
        return pltpu.make_async_copy(xws_hbm.at[sidx_v.at[j]],
                                     rows_v.at[b], gsem[b])

    def scat_start(j, b):
        pltpu.async_copy(rows_v.at[b], agg_sh.at[didx_v.at[j]], ssem[b],
                         add=True)

    def scat_wait(j, b):
        pltpu.make_async_copy(rows_v.at[b], agg_sh.at[didx_v.at[j]],
                              ssem[b]).wait()

    def sup(k, _):
        pltpu.sync_copy(src_hbm.at[wid, k], sidx_v)
        pltpu.sync_copy(dst_hbm.at[wid, k], didx_v)

        for b in range(NBUF):
            gath(b, b).start()
        for b in range(NBUF):
            gath(b, b).wait()
            scat_start(b, b)

        def it(i, _):
            for b in range(NBUF):
                j = NBUF * i + b
                scat_wait(j - NBUF, b)
                gath(j, b).start()
            for b in range(NBUF):
                j = NBUF * i + b
                gath(j, b).wait()
                scat_start(j, b)
            return 0

        lax.fori_loop(1, SUB // NBUF, it, 0)

        for b in range(NBUF):
            scat_wait(SUB - NBUF + b, b)
        return 0

    lax.fori_loop(0, NSUP, sup, 0)
    plsc.subcore_barrier()

    @pl.when(c == 0)
    def _():
        pltpu.sync_copy(agg_sh.at[pl.ds(s * SLAB, SLAB)],
                        out0_hbm.at[pl.ds(s * SLAB, SLAB)])

    @pl.when(c == 1)
    def _():
        pltpu.sync_copy(agg_sh.at[pl.ds(s * SLAB, SLAB)],
                        out1_hbm.at[pl.ds(s * SLAB, SLAB)])


def _k3_body(nf_ref, f_ref, a0_ref, a1_ref, xws_ref, dinv_ref,
             wca_ref, wcb_ref, bc_ref, bgc_ref, out_ref):
    dinv = dinv_ref[...]
    gc = (a0_ref[...] + a1_ref[...] + xws_ref[...]) * dinv + bgc_ref[...]
    h = (jnp.dot(nf_ref[...], wca_ref[...], preferred_element_type=jnp.float32)
         + jnp.dot(gc, wcb_ref[...], preferred_element_type=jnp.float32)
         + bc_ref[...])
    out_ref[:, :D] = jnp.where(h > 0, h, jnp.exp(h) - 1.0)
    out_ref[:, D:] = f_ref[...]


_k3 = pl.pallas_call(
    _k3_body,
    grid=(NB,),
    in_specs=[
        pl.BlockSpec((ROWS, D), lambda i: (i, 0)),
        pl.BlockSpec((ROWS, D), lambda i: (i, 0)),
        pl.BlockSpec((ROWS, D), lambda i: (i, 0)),
        pl.BlockSpec((ROWS, D), lambda i: (i, 0)),
        pl.BlockSpec((ROWS, D), lambda i: (i, 0)),
        pl.BlockSpec((ROWS, 1), lambda i: (i, 0)),
        pl.BlockSpec((D, D), lambda i: (0, 0)),
        pl.BlockSpec((D, D), lambda i: (0, 0)),
        pl.BlockSpec((1, D), lambda i: (0, 0)),
        pl.BlockSpec((1, D), lambda i: (0, 0)),
    ],
    out_specs=pl.BlockSpec((ROWS, 2 * D), lambda i: (i, 0)),
    out_shape=jax.ShapeDtypeStruct((N, 2 * D), jnp.float32),
)


def kernel(feats, edges, batch, W1, b1, Wgc, bgc, Wc, bc):
    src4d = edges[0].reshape(NW, NSUP, SUB, C)
    dst4d = edges[1].reshape(NW, NSUP, SUB, C)

    nfeats, xw = _k1(feats, W1, b1.reshape(1, D), Wgc[:D], Wgc[D:])

    deg0, deg1 = _sc_deg(dst4d)
    d0 = deg0[:N].reshape(N, 1)
    d1 = deg1[:N].reshape(N, 1)

    xws, dinv = _k2(d0, d1, xw)

    zeros = jnp.zeros((NPAD, D), jnp.float32)
    agg0, agg1 = _sc_agg(src4d, dst4d, xws, zeros)

    out_feats = _k3(nfeats, feats, agg0[:N], agg1[:N], xws, dinv,
                    Wc[:D], Wc[D:], bc.reshape(1, D), bgc.reshape(1, D))
    return (out_feats, edges, batch)


# speedup vs baseline: 1.0902x; 1.0902x over previous
import functools

import jax
import jax.numpy as jnp
from jax import lax
from jax.experimental import pallas as pl
from jax.experimental.pallas import tpu as pltpu
from jax.experimental.pallas import tpu_sc as plsc

N = 10000
E = 320000
D = 128

ROWS = 1000
NB = N // ROWS

NC = 2
NS = 16
NW = NC * NS
EW = E // NW
C = 100
NCHUNK = EW // C
SUB = 25
NSUP = NCHUNK // SUB
NPAD = 10240
SLAB = NPAD // NS

_MESH = plsc.VectorSubcoreMesh(core_axis_name="c", subcore_axis_name="s")


def _k1_body(f_ref, w1_ref, b1_ref, wga_ref, wgb_ref, nf_ref, xw_ref):
    f = f_ref[...]
    h = jnp.dot(f, w1_ref[...], preferred_element_type=jnp.float32) + b1_ref[...]
    nf = jnp.where(h > 0, h, jnp.exp(h) - 1.0)
    nf_ref[...] = nf
    xw_ref[...] = (jnp.dot(nf, wga_ref[...], preferred_element_type=jnp.float32)
                   + jnp.dot(f, wgb_ref[...], preferred_element_type=jnp.float32))


_k1 = pl.pallas_call(
    _k1_body,
    grid=(NB,),
    in_specs=[
        pl.BlockSpec((ROWS, D), lambda i: (i, 0)),
        pl.BlockSpec((D, D), lambda i: (0, 0)),
        pl.BlockSpec((1, D), lambda i: (0, 0)),
        pl.BlockSpec((D, D), lambda i: (0, 0)),
        pl.BlockSpec((D, D), lambda i: (0, 0)),
    ],
    out_specs=[
        pl.BlockSpec((ROWS, D), lambda i: (i, 0)),
        pl.BlockSpec((ROWS, D), lambda i: (i, 0)),
    ],
    out_shape=[
        jax.ShapeDtypeStruct((N, D), jnp.float32),
        jax.ShapeDtypeStruct((N, D), jnp.float32),
    ],
)


@functools.partial(
    pl.kernel,
    out_type=[jax.ShapeDtypeStruct((NPAD,), jnp.float32),
              jax.ShapeDtypeStruct((NPAD,), jnp.float32)],
    mesh=_MESH,
    scratch_types=[
        pltpu.VMEM((SUB, C), jnp.int32),
        pltpu.VMEM((112,), jnp.float32),
        pltpu.VMEM((SLAB,), jnp.float32),
        pltpu.VMEM_SHARED((NPAD,), jnp.float32),
    ],
)
def _sc_deg(dst_hbm, out0_hbm, out1_hbm, didx_v, ones_v, zbuf_v, deg_sh):
    c = lax.axis_index("c")
    s = lax.axis_index("s")
    wid = c * NS + s

    def fill_ones(i, _):
        ones_v[pl.ds(i * 16, 16)] = jnp.full((16,), 1.0, jnp.float32)
        return 0

    lax.fori_loop(0, 112 // 16, fill_ones, 0)

    def fill_zero(i, _):
        zbuf_v[pl.ds(i * 16, 16)] = jnp.zeros((16,), jnp.float32)
        return 0

    lax.fori_loop(0, SLAB // 16, fill_zero, 0)

    pltpu.sync_copy(zbuf_v, deg_sh.at[pl.ds(s * SLAB, SLAB)])
    plsc.subcore_barrier()

    def sup(k, _):
        pltpu.sync_copy(dst_hbm.at[wid, k], didx_v)

        def body(j, _):
            pltpu.sync_copy(ones_v.at[pl.ds(0, C)], deg_sh.at[didx_v.at[j]],
                            add=True)
            return 0

        lax.fori_loop(0, SUB, body, 0)
        return 0

    lax.fori_loop(0, NSUP, sup, 0)
    plsc.subcore_barrier()

    @pl.when(c == 0)
    def _():
        pltpu.sync_copy(deg_sh.at[pl.ds(s * SLAB, SLAB)],
                        out0_hbm.at[pl.ds(s * SLAB, SLAB)])

    @pl.when(c == 1)
    def _():
        pltpu.sync_copy(deg_sh.at[pl.ds(s * SLAB, SLAB)],
                        out1_hbm.at[pl.ds(s * SLAB, SLAB)])


def _k2_body(d0_ref, d1_ref, xw_ref, xws_ref, dinv_ref):
    deg = d0_ref[...] + d1_ref[...] + 1.0
    dinv = lax.rsqrt(deg)
    dinv_ref[...] = dinv
    xws_ref[...] = xw_ref[...] * dinv


_k2 = pl.pallas_call(
    _k2_body,
    grid=(NB,),
    in_specs=[
        pl.BlockSpec((ROWS, 1), lambda i: (i, 0)),
        pl.BlockSpec((ROWS, 1), lambda i: (i, 0)),
        pl.BlockSpec((ROWS, D), lambda i: (i, 0)),
    ],
    out_specs=[
        pl.BlockSpec((ROWS, D), lambda i: (i, 0)),
        pl.BlockSpec((ROWS, 1), lambda i: (i, 0)),
    ],
    out_shape=[
        jax.ShapeDtypeStruct((N, D), jnp.float32),
        jax.ShapeDtypeStruct((N, 1), jnp.float32),
    ],
)


@functools.partial(
    pl.kernel,
    out_type=[jax.ShapeDtypeStruct((NPAD, D), jnp.float32),
              jax.ShapeDtypeStruct((NPAD, D), jnp.float32)],
    mesh=_MESH,
    scratch_types=[
        pltpu.VMEM((SUB, C), jnp.int32),
        pltpu.VMEM((SUB, C), jnp.int32),
        pltpu.VMEM((2, C, D), jnp.float32),
        pltpu.VMEM_SHARED((NPAD, D), jnp.float32),
        pltpu.SemaphoreType.DMA,
        pltpu.SemaphoreType.DMA,
    ],
)
def _sc_agg(src_hbm, dst_hbm, xws_hbm, zeros_hbm, out0_hbm, out1_hbm,
            sidx_v, didx_v, rows_v, agg_sh, sem0, sem1):
    c = lax.axis_index("c")
    s = lax.axis_index("s")
    wid = c * NS + s

    pltpu.sync_copy(zeros_hbm.at[pl.ds(s * SLAB, SLAB)],
                    agg_sh.at[pl.ds(s * SLAB, SLAB)])
    plsc.subcore_barrier()

    def gather(j, buf, sem):
        return pltpu.make_async_copy(xws_hbm.at[sidx_v.at[j]],
                                     rows_v.at[buf], sem)

    def scat(j, buf):
        pltpu.sync_copy(rows_v.at[buf], agg_sh.at[didx_v.at[j]], add=True)

    def sup(k, _):
        pltpu.sync_copy(src_hbm.at[wid, k], sidx_v)
        pltpu.sync_copy(dst_hbm.at[wid, k], didx_v)

        gather(0, 0, sem0).start()

        def body(j, _):
            gather(2 * j + 1, 1, sem1).start()
            gather(2 * j, 0, sem0).wait()
            scat(2 * j, 0)
            gather(2 * j + 2, 0, sem0).start()
            gather(2 * j + 1, 1, sem1).wait()
            scat(2 * j + 1, 1)
            return 0

        lax.fori_loop(0, (SUB - 1) // 2, body, 0)
        gather(SUB - 1, 0, sem0).wait()
        scat(SUB - 1, 0)
        return 0

    lax.fori_loop(0, NSUP, sup, 0)
    plsc.subcore_barrier()

    @pl.when(c == 0)
    def _():
        pltpu.sync_copy(agg_sh.at[pl.ds(s * SLAB, SLAB)],
                        out0_hbm.at[pl.ds(s * SLAB, SLAB)])

    @pl.when(c == 1)
    def _():
        pltpu.sync_copy(agg_sh.at[pl.ds(s * SLAB, SLAB)],
                        out1_hbm.at[pl.ds(s * SLAB, SLAB)])


def _k3_body(nf_ref, f_ref, a0_ref, a1_ref, xws_ref, dinv_ref,
             wca_ref, wcb_ref, bc_ref, bgc_ref, out_ref):
    dinv = dinv_ref[...]
    gc = (a0_ref[...] + a1_ref[...] + xws_ref[...]) * dinv + bgc_ref[...]
    h = (jnp.dot(nf_ref[...], wca_ref[...], preferred_element_type=jnp.float32)
         + jnp.dot(gc, wcb_ref[...], preferred_element_type=jnp.float32)
         + bc_ref[...])
    out_ref[:, :D] = jnp.where(h > 0, h, jnp.exp(h) - 1.0)
    out_ref[:, D:] = f_ref[...]


_k3 = pl.pallas_call(
    _k3_body,
    grid=(NB,),
    in_specs=[
        pl.BlockSpec((ROWS, D), lambda i: (i, 0)),
        pl.BlockSpec((ROWS, D), lambda i: (i, 0)),
        pl.BlockSpec((ROWS, D), lambda i: (i, 0)),
        pl.BlockSpec((ROWS, D), lambda i: (i, 0)),
        pl.BlockSpec((ROWS, D), lambda i: (i, 0)),
        pl.BlockSpec((ROWS, 1), lambda i: (i, 0)),
        pl.BlockSpec((D, D), lambda i: (0, 0)),
        pl.BlockSpec((D, D), lambda i: (0, 0)),
        pl.BlockSpec((1, D), lambda i: (0, 0)),
        pl.BlockSpec((1, D), lambda i: (0, 0)),
    ],
    out_specs=pl.BlockSpec((ROWS, 2 * D), lambda i: (i, 0)),
    out_shape=jax.ShapeDtypeStruct((N, 2 * D), jnp.float32),
)


def kernel(feats, edges, batch, W1, b1, Wgc, bgc, Wc, bc):
    src4d = edges[0].reshape(NW, NSUP, SUB, C)
    dst4d = edges[1].reshape(NW, NSUP, SUB, C)

    nfeats, xw = _k1(feats, W1, b1.reshape(1, D), Wgc[:D], Wgc[D:])

    deg0, deg1 = _sc_deg(dst4d)
    d0 = deg0[:N].reshape(N, 1)
    d1 = deg1[:N].reshape(N, 1)

    xws, dinv = _k2(d0, d1, xw)

    zeros = jnp.zeros((NPAD, D), jnp.float32)
    agg0, agg1 = _sc_agg(src4d, dst4d, xws, zeros)

    out_feats = _k3(nfeats, feats, agg0[:N], agg1[:N], xws, dinv,
                    Wc[:D], Wc[D:], bc.reshape(1, D), bgc.reshape(1, D))
    return (out_feats, edges, batch)


# speedup vs baseline: 1.1507x; 1.0554x over previous
import functools

import jax
import jax.numpy as jnp
from jax import lax
from jax.experimental import pallas as pl
from jax.experimental.pallas import tpu as pltpu
from jax.experimental.pallas import tpu_sc as plsc

N = 10000
E = 320000
D = 128

ROWS = 1000
NB = N // ROWS

NC = 2
NS = 16
NW = NC * NS
EW = E // NW
C = 100
NCHUNK = EW // C
SUB = 25
NSUP = NCHUNK // SUB
NPAD = 10240
SLAB = NPAD // NS

_MESH = plsc.VectorSubcoreMesh(core_axis_name="c", subcore_axis_name="s")


def _k1_body(f_ref, w1_ref, b1_ref, wga_ref, wgb_ref, d0_ref, d1_ref,
             nf_ref, xws_ref, dinv_ref):
    f = f_ref[...]
    h = jnp.dot(f, w1_ref[...], preferred_element_type=jnp.float32) + b1_ref[...]
    nf = jnp.where(h > 0, h, jnp.exp(h) - 1.0)
    nf_ref[...] = nf
    xw = (jnp.dot(nf, wga_ref[...], preferred_element_type=jnp.float32)
          + jnp.dot(f, wgb_ref[...], preferred_element_type=jnp.float32))
    deg = d0_ref[...] + d1_ref[...] + 1.0
    dinv = lax.rsqrt(deg)
    dinv_ref[...] = dinv
    xws_ref[...] = xw * dinv


_k1 = pl.pallas_call(
    _k1_body,
    grid=(NB,),
    in_specs=[
        pl.BlockSpec((ROWS, D), lambda i: (i, 0)),
        pl.BlockSpec((D, D), lambda i: (0, 0)),
        pl.BlockSpec((1, D), lambda i: (0, 0)),
        pl.BlockSpec((D, D), lambda i: (0, 0)),
        pl.BlockSpec((D, D), lambda i: (0, 0)),
        pl.BlockSpec((ROWS, 1), lambda i: (i, 0)),
        pl.BlockSpec((ROWS, 1), lambda i: (i, 0)),
    ],
    out_specs=[
        pl.BlockSpec((ROWS, D), lambda i: (i, 0)),
        pl.BlockSpec((ROWS, D), lambda i: (i, 0)),
        pl.BlockSpec((ROWS, 1), lambda i: (i, 0)),
    ],
    out_shape=[
        jax.ShapeDtypeStruct((N, D), jnp.float32),
        jax.ShapeDtypeStruct((N, D), jnp.float32),
        jax.ShapeDtypeStruct((N, 1), jnp.float32),
    ],
)


@functools.partial(
    pl.kernel,
    out_type=[jax.ShapeDtypeStruct((NPAD,), jnp.float32),
              jax.ShapeDtypeStruct((NPAD,), jnp.float32)],
    mesh=_MESH,
    scratch_types=[
        pltpu.VMEM((SUB, C), jnp.int32),
        pltpu.VMEM((112,), jnp.float32),
        pltpu.VMEM((SLAB,), jnp.float32),
        pltpu.VMEM_SHARED((NPAD,), jnp.float32),
    ],
)
def _sc_deg(dst_hbm, out0_hbm, out1_hbm, didx_v, ones_v, zbuf_v, deg_sh):
    c = lax.axis_index("c")
    s = lax.axis_index("s")
    wid = c * NS + s

    def fill_ones(i, _):
        ones_v[pl.ds(i * 16, 16)] = jnp.full((16,), 1.0, jnp.float32)
        return 0

    lax.fori_loop(0, 112 // 16, fill_ones, 0)

    def fill_zero(i, _):
        zbuf_v[pl.ds(i * 16, 16)] = jnp.zeros((16,), jnp.float32)
        return 0

    lax.fori_loop(0, SLAB // 16, fill_zero, 0)

    pltpu.sync_copy(zbuf_v, deg_sh.at[pl.ds(s * SLAB, SLAB)])
    plsc.subcore_barrier()

    def sup(k, _):
        pltpu.sync_copy(dst_hbm.at[wid, k], didx_v)

        def body(j, _):
            pltpu.sync_copy(ones_v.at[pl.ds(0, C)], deg_sh.at[didx_v.at[j]],
                            add=True)
            return 0

        lax.fori_loop(0, SUB, body, 0)
        return 0

    lax.fori_loop(0, NSUP, sup, 0)
    plsc.subcore_barrier()

    @pl.when(c == 0)
    def _():
        pltpu.sync_copy(deg_sh.at[pl.ds(s * SLAB, SLAB)],
                        out0_hbm.at[pl.ds(s * SLAB, SLAB)])

    @pl.when(c == 1)
    def _():
        pltpu.sync_copy(deg_sh.at[pl.ds(s * SLAB, SLAB)],
                        out1_hbm.at[pl.ds(s * SLAB, SLAB)])


@functools.partial(
    pl.kernel,
    out_type=[jax.ShapeDtypeStruct((NPAD, D), jnp.float32),
              jax.ShapeDtypeStruct((NPAD, D), jnp.float32)],
    mesh=_MESH,
    scratch_types=[
        pltpu.VMEM((SUB, C), jnp.int32),
        pltpu.VMEM((SUB, C), jnp.int32),
        pltpu.VMEM((2, C, D), jnp.float32),
        pltpu.VMEM_SHARED((NPAD, D), jnp.float32),
        pltpu.SemaphoreType.DMA,
        pltpu.SemaphoreType.DMA,
    ],
)
def _sc_agg(src_hbm, dst_hbm, xws_hbm, out0_hbm, out1_hbm,
            sidx_v, didx_v, rows_v, agg_sh, sem0, sem1):
    c = lax.axis_index("c")
    s = lax.axis_index("s")
    wid = c * NS + s

    def fill_zero(i, _):
        rows_v[0, i // 8, pl.ds((i % 8) * 16, 16)] = jnp.zeros((16,),
                                                               jnp.float32)
        return 0

    lax.fori_loop(0, 80 * 8, fill_zero, 0)

    def zinit(i, _):
        pltpu.sync_copy(rows_v.at[0, pl.ds(0, 80)],
                        agg_sh.at[pl.ds(s * SLAB + i * 80, 80)])
        return 0

    lax.fori_loop(0, SLAB // 80, zinit, 0)
    plsc.subcore_barrier()

    def gather(j, buf, sem):
        return pltpu.make_async_copy(xws_hbm.at[sidx_v.at[j]],
                                     rows_v.at[buf], sem)

    def scat(j, buf):
        pltpu.sync_copy(rows_v.at[buf], agg_sh.at[didx_v.at[j]], add=True)

    def sup(k, _):
        pltpu.sync_copy(src_hbm.at[wid, k], sidx_v)
        pltpu.sync_copy(dst_hbm.at[wid, k], didx_v)

        gather(0, 0, sem0).start()

        def body(j, _):
            gather(2 * j + 1, 1, sem1).start()
            gather(2 * j, 0, sem0).wait()
            scat(2 * j, 0)
            gather(2 * j + 2, 0, sem0).start()
            gather(2 * j + 1, 1, sem1).wait()
            scat(2 * j + 1, 1)
            return 0

        lax.fori_loop(0, (SUB - 1) // 2, body, 0)
        gather(SUB - 1, 0, sem0).wait()
        scat(SUB - 1, 0)
        return 0

    lax.fori_loop(0, NSUP, sup, 0)
    plsc.subcore_barrier()

    @pl.when(c == 0)
    def _():
        pltpu.sync_copy(agg_sh.at[pl.ds(s * SLAB, SLAB)],
                        out0_hbm.at[pl.ds(s * SLAB, SLAB)])

    @pl.when(c == 1)
    def _():
        pltpu.sync_copy(agg_sh.at[pl.ds(s * SLAB, SLAB)],
                        out1_hbm.at[pl.ds(s * SLAB, SLAB)])


def _k3_body(nf_ref, f_ref, a0_ref, a1_ref, xws_ref, dinv_ref,
             wca_ref, wcb_ref, bc_ref, bgc_ref, out_ref):
    dinv = dinv_ref[...]
    gc = (a0_ref[...] + a1_ref[...] + xws_ref[...]) * dinv + bgc_ref[...]
    h = (jnp.dot(nf_ref[...], wca_ref[...], preferred_element_type=jnp.float32)
         + jnp.dot(gc, wcb_ref[...], preferred_element_type=jnp.float32)
         + bc_ref[...])
    out_ref[:, :D] = jnp.where(h > 0, h, jnp.exp(h) - 1.0)
    out_ref[:, D:] = f_ref[...]


_k3 = pl.pallas_call(
    _k3_body,
    grid=(NB,),
    in_specs=[
        pl.BlockSpec((ROWS, D), lambda i: (i, 0)),
        pl.BlockSpec((ROWS, D), lambda i: (i, 0)),
        pl.BlockSpec((ROWS, D), lambda i: (i, 0)),
        pl.BlockSpec((ROWS, D), lambda i: (i, 0)),
        pl.BlockSpec((ROWS, D), lambda i: (i, 0)),
        pl.BlockSpec((ROWS, 1), lambda i: (i, 0)),
        pl.BlockSpec((D, D), lambda i: (0, 0)),
        pl.BlockSpec((D, D), lambda i: (0, 0)),
        pl.BlockSpec((1, D), lambda i: (0, 0)),
        pl.BlockSpec((1, D), lambda i: (0, 0)),
    ],
    out_specs=pl.BlockSpec((ROWS, 2 * D), lambda i: (i, 0)),
    out_shape=jax.ShapeDtypeStruct((N, 2 * D), jnp.float32),
)


def kernel(feats, edges, batch, W1, b1, Wgc, bgc, Wc, bc):
    src4d = edges[0].reshape(NW, NSUP, SUB, C)
    dst4d = edges[1].reshape(NW, NSUP, SUB, C)

    deg0, deg1 = _sc_deg(dst4d)

    nfeats, xws, dinv = _k1(feats, W1, b1.reshape(1, D), Wgc[:D], Wgc[D:],
                            deg0.reshape(NPAD, 1), deg1.reshape(NPAD, 1))

    agg0, agg1 = _sc_agg(src4d, dst4d, xws)

    out_feats = _k3(nfeats, feats, agg0, agg1, xws, dinv,
                    Wc[:D], Wc[D:], bc.reshape(1, D), bgc.reshape(1, D))
    return (out_feats, edges, batch)


# speedup vs baseline: 1.2312x; 1.0700x over previous
import functools

import jax
import jax.numpy as jnp
from jax import lax
from jax.experimental import pallas as pl
from jax.experimental.pallas import tpu as pltpu
from jax.experimental.pallas import tpu_sc as plsc

N = 10000
E = 320000
D = 128

ROWS = 1000
NB = N // ROWS

NC = 2
NS = 16
NW = NC * NS
EW = E // NW
C = 100
NCHUNK = EW // C
SUB = 25
NSUP = NCHUNK // SUB
NPAD = 10240
SLAB = NPAD // NS

_MESH = plsc.VectorSubcoreMesh(core_axis_name="c", subcore_axis_name="s")


def _k1_body(f_ref, w1_ref, b1_ref, wga_ref, wgb_ref, d0_ref, d1_ref,
             nf_ref, xws_ref, dinv_ref):
    f = f_ref[...]
    h = jnp.dot(f, w1_ref[...], preferred_element_type=jnp.float32) + b1_ref[...]
    nf = jnp.where(h > 0, h, jnp.exp(h) - 1.0)
    nf_ref[...] = nf
    xw = (jnp.dot(nf, wga_ref[...], preferred_element_type=jnp.float32)
          + jnp.dot(f, wgb_ref[...], preferred_element_type=jnp.float32))
    deg = d0_ref[...] + d1_ref[...] + 1.0
    dinv = lax.rsqrt(deg)
    dinv_ref[...] = dinv
    xws_ref[...] = xw * dinv


_k1 = pl.pallas_call(
    _k1_body,
    grid=(NB,),
    in_specs=[
        pl.BlockSpec((ROWS, D), lambda i: (i, 0)),
        pl.BlockSpec((D, D), lambda i: (0, 0)),
        pl.BlockSpec((1, D), lambda i: (0, 0)),
        pl.BlockSpec((D, D), lambda i: (0, 0)),
        pl.BlockSpec((D, D), lambda i: (0, 0)),
        pl.BlockSpec((ROWS, 1), lambda i: (i, 0)),
        pl.BlockSpec((ROWS, 1), lambda i: (i, 0)),
    ],
    out_specs=[
        pl.BlockSpec((ROWS, D), lambda i: (i, 0)),
        pl.BlockSpec((ROWS, D), lambda i: (i, 0)),
        pl.BlockSpec((ROWS, 1), lambda i: (i, 0)),
    ],
    out_shape=[
        jax.ShapeDtypeStruct((N, D), jnp.float32),
        jax.ShapeDtypeStruct((N, D), jnp.float32),
        jax.ShapeDtypeStruct((N, 1), jnp.float32),
    ],
)


@functools.partial(
    pl.kernel,
    out_type=[jax.ShapeDtypeStruct((NPAD,), jnp.float32),
              jax.ShapeDtypeStruct((NPAD,), jnp.float32)],
    mesh=_MESH,
    scratch_types=[
        pltpu.VMEM((SUB, C), jnp.int32),
        pltpu.VMEM((112,), jnp.float32),
        pltpu.VMEM((SLAB,), jnp.float32),
        pltpu.VMEM_SHARED((NPAD,), jnp.float32),
    ],
)
def _sc_deg(dst_hbm, out0_hbm, out1_hbm, didx_v, ones_v, zbuf_v, deg_sh):
    c = lax.axis_index("c")
    s = lax.axis_index("s")
    wid = c * NS + s

    def fill_ones(i, _):
        ones_v[pl.ds(i * 16, 16)] = jnp.full((16,), 1.0, jnp.float32)
        return 0

    lax.fori_loop(0, 112 // 16, fill_ones, 0)

    def fill_zero(i, _):
        zbuf_v[pl.ds(i * 16, 16)] = jnp.zeros((16,), jnp.float32)
        return 0

    lax.fori_loop(0, SLAB // 16, fill_zero, 0)

    pltpu.sync_copy(zbuf_v, deg_sh.at[pl.ds(s * SLAB, SLAB)])
    plsc.subcore_barrier()

    def sup(k, _):
        pltpu.sync_copy(dst_hbm.at[wid, k], didx_v)

        def body(j, _):
            pltpu.sync_copy(ones_v.at[pl.ds(0, C)], deg_sh.at[didx_v.at[j]],
                            add=True)
            return 0

        lax.fori_loop(0, SUB, body, 0)
        return 0

    lax.fori_loop(0, NSUP, sup, 0)
    plsc.subcore_barrier()

    @pl.when(c == 0)
    def _():
        pltpu.sync_copy(deg_sh.at[pl.ds(s * SLAB, SLAB)],
                        out0_hbm.at[pl.ds(s * SLAB, SLAB)])

    @pl.when(c == 1)
    def _():
        pltpu.sync_copy(deg_sh.at[pl.ds(s * SLAB, SLAB)],
                        out1_hbm.at[pl.ds(s * SLAB, SLAB)])


@functools.partial(
    pl.kernel,
    out_type=[jax.ShapeDtypeStruct((NPAD, D), jnp.float32),
              jax.ShapeDtypeStruct((NPAD, D), jnp.float32)],
    mesh=_MESH,
    scratch_types=[
        pltpu.VMEM((SUB, C), jnp.int32),
        pltpu.VMEM((SUB, C), jnp.int32),
        pltpu.VMEM((3, C, D), jnp.float32),
        pltpu.VMEM_SHARED((NPAD, D), jnp.float32),
        pltpu.SemaphoreType.DMA,
        pltpu.SemaphoreType.DMA,
        pltpu.SemaphoreType.DMA,
    ],
)
def _sc_agg(src_hbm, dst_hbm, xws_hbm, out0_hbm, out1_hbm,
            sidx_v, didx_v, rows_v, agg_sh, sem0, sem1, sem2):
    c = lax.axis_index("c")
    s = lax.axis_index("s")
    wid = c * NS + s

    def fill_zero(i, _):
        rows_v[0, i // 8, pl.ds((i % 8) * 16, 16)] = jnp.zeros((16,),
                                                               jnp.float32)
        return 0

    lax.fori_loop(0, 80 * 8, fill_zero, 0)

    def zinit(i, _):
        pltpu.sync_copy(rows_v.at[0, pl.ds(0, 80)],
                        agg_sh.at[pl.ds(s * SLAB + i * 80, 80)])
        return 0

    lax.fori_loop(0, SLAB // 80, zinit, 0)
    plsc.subcore_barrier()

    sems = (sem0, sem1, sem2)

    def gather(j, buf):
        return pltpu.make_async_copy(xws_hbm.at[sidx_v.at[j]],
                                     rows_v.at[buf], sems[buf])

    def scat(j, buf):
        pltpu.sync_copy(rows_v.at[buf], agg_sh.at[didx_v.at[j]], add=True)

    def sup(k, _):
        pltpu.sync_copy(src_hbm.at[wid, k], sidx_v)
        pltpu.sync_copy(dst_hbm.at[wid, k], didx_v)

        gather(0, 0).start()
        gather(1, 1).start()
        for j in range(SUB):
            if j + 2 < SUB:
                gather(j + 2, (j + 2) % 3).start()
            gather(j, j % 3).wait()
            scat(j, j % 3)
        return 0

    lax.fori_loop(0, NSUP, sup, 0)
    plsc.subcore_barrier()

    @pl.when(c == 0)
    def _():
        pltpu.sync_copy(agg_sh.at[pl.ds(s * SLAB, SLAB)],
                        out0_hbm.at[pl.ds(s * SLAB, SLAB)])

    @pl.when(c == 1)
    def _():
        pltpu.sync_copy(agg_sh.at[pl.ds(s * SLAB, SLAB)],
                        out1_hbm.at[pl.ds(s * SLAB, SLAB)])


def _k3_body(nf_ref, f_ref, a0_ref, a1_ref, xws_ref, dinv_ref,
             wca_ref, wcb_ref, bc_ref, bgc_ref, out_ref):
    dinv = dinv_ref[...]
    gc = (a0_ref[...] + a1_ref[...] + xws_ref[...]) * dinv + bgc_ref[...]
    h = (jnp.dot(nf_ref[...], wca_ref[...], preferred_element_type=jnp.float32)
         + jnp.dot(gc, wcb_ref[...], preferred_element_type=jnp.float32)
         + bc_ref[...])
    out_ref[:, :D] = jnp.where(h > 0, h, jnp.exp(h) - 1.0)
    out_ref[:, D:] = f_ref[...]


_k3 = pl.pallas_call(
    _k3_body,
    grid=(NB,),
    in_specs=[
        pl.BlockSpec((ROWS, D), lambda i: (i, 0)),
        pl.BlockSpec((ROWS, D), lambda i: (i, 0)),
        pl.BlockSpec((ROWS, D), lambda i: (i, 0)),
        pl.BlockSpec((ROWS, D), lambda i: (i, 0)),
        pl.BlockSpec((ROWS, D), lambda i: (i, 0)),
        pl.BlockSpec((ROWS, 1), lambda i: (i, 0)),
        pl.BlockSpec((D, D), lambda i: (0, 0)),
        pl.BlockSpec((D, D), lambda i: (0, 0)),
        pl.BlockSpec((1, D), lambda i: (0, 0)),
        pl.BlockSpec((1, D), lambda i: (0, 0)),
    ],
    out_specs=pl.BlockSpec((ROWS, 2 * D), lambda i: (i, 0)),
    out_shape=jax.ShapeDtypeStruct((N, 2 * D), jnp.float32),
)


def kernel(feats, edges, batch, W1, b1, Wgc, bgc, Wc, bc):
    src4d = edges[0].reshape(NW, NSUP, SUB, C)
    dst4d = edges[1].reshape(NW, NSUP, SUB, C)

    deg0, deg1 = _sc_deg(dst4d)

    nfeats, xws, dinv = _k1(feats, W1, b1.reshape(1, D), Wgc[:D], Wgc[D:],
                            deg0.reshape(NPAD, 1), deg1.reshape(NPAD, 1))

    agg0, agg1 = _sc_agg(src4d, dst4d, xws)

    out_feats = _k3(nfeats, feats, agg0, agg1, xws, dinv,
                    Wc[:D], Wc[D:], bc.reshape(1, D), bgc.reshape(1, D))
    return (out_feats, edges, batch)


# speedup vs baseline: 1.3039x; 1.0590x over previous
import functools

import jax
import jax.numpy as jnp
from jax import lax
from jax.experimental import pallas as pl
from jax.experimental.pallas import tpu as pltpu
from jax.experimental.pallas import tpu_sc as plsc

N = 10000
E = 320000
D = 128

ROWS = 1000
NB = N // ROWS

NC = 2
NS = 16
NW = NC * NS
EW = E // NW
C = 100
NCHUNK = EW // C
SUB = 25
NSUP = NCHUNK // SUB
NPAD = 10240
SLAB = NPAD // NS

_MESH = plsc.VectorSubcoreMesh(core_axis_name="c", subcore_axis_name="s")


def _k1_body(f_ref, w1_ref, b1_ref, wga_ref, wgb_ref, d0_ref, d1_ref,
             nf_ref, xws_ref, dinv_ref):
    f = f_ref[...]
    h = jnp.dot(f, w1_ref[...], preferred_element_type=jnp.float32) + b1_ref[...]
    nf = jnp.where(h > 0, h, jnp.exp(h) - 1.0)
    nf_ref[...] = nf
    xw = (jnp.dot(nf, wga_ref[...], preferred_element_type=jnp.float32)
          + jnp.dot(f, wgb_ref[...], preferred_element_type=jnp.float32))
    deg = d0_ref[...] + d1_ref[...] + 1.0
    dinv = lax.rsqrt(deg)
    dinv_ref[...] = dinv
    xws_ref[...] = xw * dinv


_k1 = pl.pallas_call(
    _k1_body,
    grid=(NB,),
    in_specs=[
        pl.BlockSpec((ROWS, D), lambda i: (i, 0)),
        pl.BlockSpec((D, D), lambda i: (0, 0)),
        pl.BlockSpec((1, D), lambda i: (0, 0)),
        pl.BlockSpec((D, D), lambda i: (0, 0)),
        pl.BlockSpec((D, D), lambda i: (0, 0)),
        pl.BlockSpec((ROWS, 1), lambda i: (i, 0)),
        pl.BlockSpec((ROWS, 1), lambda i: (i, 0)),
    ],
    out_specs=[
        pl.BlockSpec((ROWS, D), lambda i: (i, 0)),
        pl.BlockSpec((ROWS, D), lambda i: (i, 0)),
        pl.BlockSpec((ROWS, 1), lambda i: (i, 0)),
    ],
    out_shape=[
        jax.ShapeDtypeStruct((N, D), jnp.float32),
        jax.ShapeDtypeStruct((N, D), jnp.float32),
        jax.ShapeDtypeStruct((N, 1), jnp.float32),
    ],
)


@functools.partial(
    pl.kernel,
    out_type=[jax.ShapeDtypeStruct((NPAD,), jnp.float32),
              jax.ShapeDtypeStruct((NPAD,), jnp.float32)],
    mesh=_MESH,
    scratch_types=[
        pltpu.VMEM((SUB, C), jnp.int32),
        pltpu.VMEM((112,), jnp.float32),
        pltpu.VMEM((SLAB,), jnp.float32),
        pltpu.VMEM_SHARED((NPAD,), jnp.float32),
    ],
)
def _sc_deg(e5_hbm, out0_hbm, out1_hbm, didx_v, ones_v, zbuf_v, deg_sh):
    c = lax.axis_index("c")
    s = lax.axis_index("s")
    wid = c * NS + s

    def fill_ones(i, _):
        ones_v[pl.ds(i * 16, 16)] = jnp.full((16,), 1.0, jnp.float32)
        return 0

    lax.fori_loop(0, 112 // 16, fill_ones, 0)

    def fill_zero(i, _):
        zbuf_v[pl.ds(i * 16, 16)] = jnp.zeros((16,), jnp.float32)
        return 0

    lax.fori_loop(0, SLAB // 16, fill_zero, 0)

    pltpu.sync_copy(zbuf_v, deg_sh.at[pl.ds(s * SLAB, SLAB)])
    plsc.subcore_barrier()

    def sup(k, _):
        pltpu.sync_copy(e5_hbm.at[1, wid, k], didx_v)

        def body(j, _):
            pltpu.sync_copy(ones_v.at[pl.ds(0, C)], deg_sh.at[didx_v.at[j]],
                            add=True)
            return 0

        lax.fori_loop(0, SUB, body, 0)
        return 0

    lax.fori_loop(0, NSUP, sup, 0)
    plsc.subcore_barrier()

    @pl.when(c == 0)
    def _():
        pltpu.sync_copy(deg_sh.at[pl.ds(s * SLAB, SLAB)],
                        out0_hbm.at[pl.ds(s * SLAB, SLAB)])

    @pl.when(c == 1)
    def _():
        pltpu.sync_copy(deg_sh.at[pl.ds(s * SLAB, SLAB)],
                        out1_hbm.at[pl.ds(s * SLAB, SLAB)])


@functools.partial(
    pl.kernel,
    out_type=[jax.ShapeDtypeStruct((NPAD, D), jnp.float32),
              jax.ShapeDtypeStruct((NPAD, D), jnp.float32)],
    mesh=_MESH,
    scratch_types=[
        pltpu.VMEM((SUB, C), jnp.int32),
        pltpu.VMEM((SUB, C), jnp.int32),
        pltpu.VMEM((3, C, D), jnp.float32),
        pltpu.VMEM_SHARED((NPAD, D), jnp.float32),
        pltpu.SemaphoreType.DMA,
        pltpu.SemaphoreType.DMA,
        pltpu.SemaphoreType.DMA,
    ],
)
def _sc_agg(e5_hbm, xws_hbm, out0_hbm, out1_hbm,
            sidx_v, didx_v, rows_v, agg_sh, sem0, sem1, sem2):
    c = lax.axis_index("c")
    s = lax.axis_index("s")
    wid = c * NS + s

    def fill_zero(i, _):
        rows_v[0, i // 8, pl.ds((i % 8) * 16, 16)] = jnp.zeros((16,),
                                                               jnp.float32)
        return 0

    lax.fori_loop(0, 80 * 8, fill_zero, 0)

    def zinit(i, _):
        pltpu.sync_copy(rows_v.at[0, pl.ds(0, 80)],
                        agg_sh.at[pl.ds(s * SLAB + i * 80, 80)])
        return 0

    lax.fori_loop(0, SLAB // 80, zinit, 0)
    plsc.subcore_barrier()

    sems = (sem0, sem1, sem2)

    def gather(j, buf):
        return pltpu.make_async_copy(xws_hbm.at[sidx_v.at[j]],
                                     rows_v.at[buf], sems[buf])

    def scat(j, buf):
        pltpu.sync_copy(rows_v.at[buf], agg_sh.at[didx_v.at[j]], add=True)

    def sup(k, _):
        pltpu.sync_copy(e5_hbm.at[0, wid, k], sidx_v)
        pltpu.sync_copy(e5_hbm.at[1, wid, k], didx_v)

        gather(0, 0).start()
        gather(1, 1).start()
        for j in range(SUB):
            if j + 2 < SUB:
                gather(j + 2, (j + 2) % 3).start()
            gather(j, j % 3).wait()
            scat(j, j % 3)
        return 0

    lax.fori_loop(0, NSUP, sup, 0)
    plsc.subcore_barrier()

    @pl.when(c == 0)
    def _():
        pltpu.sync_copy(agg_sh.at[pl.ds(s * SLAB, SLAB)],
                        out0_hbm.at[pl.ds(s * SLAB, SLAB)])

    @pl.when(c == 1)
    def _():
        pltpu.sync_copy(agg_sh.at[pl.ds(s * SLAB, SLAB)],
                        out1_hbm.at[pl.ds(s * SLAB, SLAB)])


def _k3_body(nf_ref, f_ref, a0_ref, a1_ref, xws_ref, dinv_ref,
             wca_ref, wcb_ref, bc_ref, bgc_ref, out_ref):
    dinv = dinv_ref[...]
    gc = (a0_ref[...] + a1_ref[...] + xws_ref[...]) * dinv + bgc_ref[...]
    h = (jnp.dot(nf_ref[...], wca_ref[...], preferred_element_type=jnp.float32)
         + jnp.dot(gc, wcb_ref[...], preferred_element_type=jnp.float32)
         + bc_ref[...])
    out_ref[:, :D] = jnp.where(h > 0, h, jnp.exp(h) - 1.0)
    out_ref[:, D:] = f_ref[...]


_k3 = pl.pallas_call(
    _k3_body,
    grid=(NB,),
    in_specs=[
        pl.BlockSpec((ROWS, D), lambda i: (i, 0)),
        pl.BlockSpec((ROWS, D), lambda i: (i, 0)),
        pl.BlockSpec((ROWS, D), lambda i: (i, 0)),
        pl.BlockSpec((ROWS, D), lambda i: (i, 0)),
        pl.BlockSpec((ROWS, D), lambda i: (i, 0)),
        pl.BlockSpec((ROWS, 1), lambda i: (i, 0)),
        pl.BlockSpec((D, D), lambda i: (0, 0)),
        pl.BlockSpec((D, D), lambda i: (0, 0)),
        pl.BlockSpec((1, D), lambda i: (0, 0)),
        pl.BlockSpec((1, D), lambda i: (0, 0)),
    ],
    out_specs=pl.BlockSpec((ROWS, 2 * D), lambda i: (i, 0)),
    out_shape=jax.ShapeDtypeStruct((N, 2 * D), jnp.float32),
)


def kernel(feats, edges, batch, W1, b1, Wgc, bgc, Wc, bc):
    e5 = edges.reshape(2, NW, NSUP, SUB, C)

    deg0, deg1 = _sc_deg(e5)

    nfeats, xws, dinv = _k1(feats, W1, b1.reshape(1, D), Wgc[:D], Wgc[D:],
                            deg0.reshape(NPAD, 1), deg1.reshape(NPAD, 1))

    agg0, agg1 = _sc_agg(e5, xws)

    out_feats = _k3(nfeats, feats, agg0, agg1, xws, dinv,
                    Wc[:D], Wc[D:], bc.reshape(1, D), bgc.reshape(1, D))
    return (out_feats, edges, batch)
